# baseline (speedup 1.0000x reference)
import functools

import jax
import jax.numpy as jnp
from jax import lax
from jax.experimental import pallas as pl
from jax.experimental.pallas import tpu as pltpu
from jax.experimental.pallas import tpu_sc as plsc


def kernel(inputs, table):
    B, S = inputs.shape
    V, D = table.shape
    idx_l = inputs[:, :128]
    idx_r = inputs[:, S - 128:]

    info = plsc.get_sparse_core_info()
    NC, NS = info.num_cores, info.num_subcores
    NW = NC * NS
    slabs_per_w = B // NW
    n_pairs = slabs_per_w // 2

    mesh = plsc.VectorSubcoreMesh(core_axis_name="c", subcore_axis_name="s")

    @functools.partial(
        pl.kernel,
        mesh=mesh,
        out_type=jax.ShapeDtypeStruct((B, S, D), jnp.float32),
        scratch_types=[
            pltpu.VMEM((slabs_per_w, 128), jnp.int32),
            pltpu.VMEM((slabs_per_w, 128), jnp.int32),
            pltpu.VMEM((2, 2, S, D), jnp.float32),
            pltpu.SemaphoreType.DMA((2,)),
            pltpu.SemaphoreType.DMA((2,)),
        ],
        compiler_params=pltpu.CompilerParams(use_tc_tiling_on_sc=False),
    )
    def gather_kernel(idxl_hbm, idxr_hbm, table_hbm, out_hbm, idx_lv, idx_rv,
                      rows_c, sem_g, sem_w):
        wid = lax.axis_index("s") * NC + lax.axis_index("c")
        slab0 = wid * slabs_per_w

        pltpu.sync_copy(idxl_hbm.at[pl.ds(slab0, slabs_per_w), :], idx_lv)
        pltpu.sync_copy(idxr_hbm.at[pl.ds(slab0, slabs_per_w), :], idx_rv)

        def fire_16(q, hs, base, vec, ks):
            for k in ks:
                pltpu.async_copy(
                    table_hbm.at[vec[k]],
                    rows_c.at[q, hs, base + k],
                    sem_g.at[q],
                )

        def fire_rows(i, q, hs):
            base = 0

            def grp(g, carry):
                vec = idx_lv[i, pl.ds(g * 16, 16)]
                fire_16(q, hs, base + g * 16, vec, range(16))
                return carry

            lax.fori_loop(0, 8, grp, 0)

            def grp_r(g, carry):
                vec = idx_rv[i, pl.ds(56 + g * 16, 16)]
                fire_16(q, hs, base + 72 + 56 + g * 16, vec, range(16))
                return carry

            lax.fori_loop(0, 4, grp_r, 0)
            vec = idx_rv[i, pl.ds(112, 16)]
            fire_16(q, hs, base + 72 + 112 - 8 + 8, vec, range(8, 16))

        def fire_pair(ii, q):
            fire_rows(2 * ii, q, 0)
            fire_rows(2 * ii + 1, q, 1)

        def drain_rows(q):
            def row(j, carry):
                pltpu.make_async_copy(
                    table_hbm.at[0], rows_c.at[q, 0, 0], sem_g.at[q]
                ).wait()
                return carry

            lax.fori_loop(0, 2 * S, row, 0)

        def wait_write(q):
            pltpu.make_async_copy(
                rows_c.at[q],
                out_hbm.at[pl.ds(slab0, 2)],
                sem_w.at[q],
            ).wait()

        fire_pair(0, 0)

        def body(g, carry):
            for p in (0, 1):
                ii = 2 * g + p
                np_ = 1 - p

                @pl.when(ii >= 1)
                def _():
                    wait_write(np_)

                @pl.when(ii + 1 < n_pairs)
                def _():
                    fire_pair(ii + 1, np_)

                drain_rows(p)
                pltpu.async_copy(
                    rows_c.at[p],
                    out_hbm.at[pl.ds(slab0 + 2 * ii, 2)],
                    sem_w.at[p],
                )
            return carry

        lax.fori_loop(0, n_pairs // 2, body, 0)
        wait_write(1)

    return gather_kernel(idx_l, idx_r, table)


# speedup vs baseline: 1.3673x; 1.3673x over previous
import functools

import jax
import jax.numpy as jnp
from jax import lax
from jax.experimental import pallas as pl
from jax.experimental.pallas import tpu as pltpu
from jax.experimental.pallas import tpu_sc as plsc


def kernel(inputs, table):
    B, S = inputs.shape
    V, D = table.shape
    idx_p = jnp.pad(inputs, ((0, 0), (0, 256 - S)))

    info = plsc.get_sparse_core_info()
    NC, NS = info.num_cores, info.num_subcores
    NW = NC * NS
    slabs_per_w = B // NW

    mesh = plsc.VectorSubcoreMesh(core_axis_name="c", subcore_axis_name="s")

    @functools.partial(
        pl.kernel,
        mesh=mesh,
        out_type=jax.ShapeDtypeStruct((B, S, D), jnp.float32),
        scratch_types=[
            pltpu.VMEM((slabs_per_w, 256), jnp.int32),
            pltpu.VMEM((2, S, D), jnp.float32),
            pltpu.SemaphoreType.DMA((2,)),
            pltpu.SemaphoreType.DMA((2,)),
        ],
    )
    def gather_kernel(idx_hbm, table_hbm, out_hbm, idx_v, rows_c,
                      sem_g, sem_w):
        wid = lax.axis_index("s") * NC + lax.axis_index("c")
        slab0 = wid * slabs_per_w

        pltpu.sync_copy(idx_hbm.at[pl.ds(slab0, slabs_per_w), :], idx_v)

        def fire_rows(i, q):
            def grp(g, carry):
                vec = idx_v[i, pl.ds(g * 16, 16)]
                for k in range(16):
                    pltpu.async_copy(
                        table_hbm.at[vec[k]],
                        rows_c.at[q, g * 16 + k],
                        sem_g.at[q],
                    )
                return carry

            lax.fori_loop(0, S // 16, grp, 0)
            vec = idx_v[i, pl.ds((S // 16) * 16, 16)]
            for k in range(S - (S // 16) * 16):
                pltpu.async_copy(
                    table_hbm.at[vec[k]],
                    rows_c.at[q, (S // 16) * 16 + k],
                    sem_g.at[q],
                )

        def drain_rows(q):
            def row(j, carry):
                pltpu.make_async_copy(
                    table_hbm.at[0], rows_c.at[q, 0], sem_g.at[q]
                ).wait()
                return carry

            lax.fori_loop(0, S, row, 0)

        def wait_write(q):
            pltpu.make_async_copy(
                rows_c.at[q], out_hbm.at[slab0], sem_w.at[q]
            ).wait()

        fire_rows(0, 0)

        def body(g, carry):
            for p in (0, 1):
                i = 2 * g + p
                np_ = 1 - p

                @pl.when(i >= 1)
                def _():
                    wait_write(np_)

                @pl.when(i + 1 < slabs_per_w)
                def _():
                    fire_rows(i + 1, np_)

                drain_rows(p)
                pltpu.async_copy(
                    rows_c.at[p], out_hbm.at[slab0 + i], sem_w.at[p]
                )
            return carry

        lax.fori_loop(0, slabs_per_w // 2, body, 0)
        wait_write(1)

    return gather_kernel(idx_p, table)
